# Initial kernel scaffold; baseline (speedup 1.0000x reference)
import jax
import jax.numpy as jnp
from jax.experimental import pallas as pl


def kernel(x, w1, b1, dw_w, dw_b, w2, b2):
    raise NotImplementedError("write your pallas kernel here")


# speedup vs baseline: 1.0379x; 1.0379x over previous
import functools

import jax
import jax.numpy as jnp
from jax.experimental import pallas as pl
from jax.experimental.pallas import tpu as pltpu


def _leff_fused(x_ref, w1_ref, b1_ref, dwk_ref, dwb_ref, w2_ref, b2_ref,
                o_ref, slab_ref, *, hh, halo, chunk):
    bt, HW, _ = x_ref.shape
    hidden = w1_ref.shape[1]
    nchunks = HW // chunk

    for b in range(bt):
        slab_ref[b, pl.ds(0, halo), :] = jnp.zeros((halo, hidden), jnp.float32)
        slab_ref[b, pl.ds(halo + HW, halo), :] = (
            jnp.zeros((halo, hidden), jnp.float32))

        for c in range(nchunks):
            q0 = c * chunk
            xc = x_ref[b, pl.ds(pl.multiple_of(q0, 8), chunk), :]
            h = jnp.dot(xc, w1_ref[...], preferred_element_type=jnp.float32)
            h = jax.nn.gelu(h + b1_ref[...], approximate=True)
            slab_ref[b, pl.ds(pl.multiple_of(halo + q0, 8), chunk), :] = h

        for c in range(nchunks):
            q0 = c * chunk
            col = (jax.lax.broadcasted_iota(jnp.int32, (chunk, 1), 0) + q0) % hh
            accL = jnp.zeros((chunk, hidden), jnp.float32)
            accC = jnp.zeros((chunk, hidden), jnp.float32)
            accR = jnp.zeros((chunk, hidden), jnp.float32)
            for dy in range(3):
                base = halo + q0 + (dy - 1) * hh
                accL += slab_ref[b, pl.ds(base - 1, chunk), :] * (
                    dwk_ref[3 * dy + 0:3 * dy + 1, :])
                accC += slab_ref[b, pl.ds(pl.multiple_of(base, 8), chunk), :] * (
                    dwk_ref[3 * dy + 1:3 * dy + 2, :])
                accR += slab_ref[b, pl.ds(base + 1, chunk), :] * (
                    dwk_ref[3 * dy + 2:3 * dy + 3, :])
            acc = accC + jnp.where(col != 0, accL, 0.0)
            acc = acc + jnp.where(col != hh - 1, accR, 0.0)
            h2 = jax.nn.gelu(acc + dwb_ref[...], approximate=True)
            y = jnp.dot(h2, w2_ref[...],
                        preferred_element_type=jnp.float32) + b2_ref[...]
            o_ref[b, pl.ds(pl.multiple_of(q0, 8), chunk), :] = (
                y.astype(o_ref.dtype))


def kernel(x, w1, b1, dw_w, dw_b, w2, b2, *, block_b=4, chunk=128):
    B, HW, dim = x.shape
    hh = int(round(HW ** 0.5))
    assert hh * hh == HW, "token count must be a perfect square"
    hidden = w1.shape[1]

    if chunk > HW or HW % chunk != 0:
        chunk = HW
    halo = -(-(hh + 1) // 8) * 8
    R = 2 * halo + HW

    block_b = max(1, min(block_b, B))
    Bp = -(-B // block_b) * block_b
    xp = jnp.pad(x, ((0, Bp - B), (0, 0), (0, 0))) if Bp != B else x

    b1r = b1.reshape(1, hidden)
    dwbr = dw_b.reshape(1, hidden)
    dwk = dw_w.reshape(hidden, 9).T
    b2r = b2.reshape(1, dim)

    kfn = functools.partial(_leff_fused, hh=hh, halo=halo, chunk=chunk)

    est_bytes = 4 * (2 * block_b * HW * (dim + dim)
                     + block_b * R * hidden
                     + 2 * (dim * hidden + hidden * dim + 12 * hidden + dim))
    vmem_limit = int(min(max(2 * est_bytes, 32 * 1024 * 1024),
                         64 * 1024 * 1024))

    cost = pl.CostEstimate(
        flops=2 * B * HW * hidden * (2 * dim) + 18 * B * HW * hidden,
        transcendentals=2 * B * HW * hidden,
        bytes_accessed=4 * (Bp * HW * 2 * dim + dim * hidden
                            + hidden * dim + 12 * hidden + dim),
    )

    out = pl.pallas_call(
        kfn,
        out_shape=jax.ShapeDtypeStruct((Bp, HW, dim), x.dtype),
        grid_spec=pltpu.PrefetchScalarGridSpec(
            num_scalar_prefetch=0,
            grid=(Bp // block_b,),
            in_specs=[
                pl.BlockSpec((block_b, HW, dim), lambda g: (g, 0, 0)),
                pl.BlockSpec((dim, hidden), lambda g: (0, 0)),
                pl.BlockSpec((1, hidden), lambda g: (0, 0)),
                pl.BlockSpec((9, hidden), lambda g: (0, 0)),
                pl.BlockSpec((1, hidden), lambda g: (0, 0)),
                pl.BlockSpec((hidden, dim), lambda g: (0, 0)),
                pl.BlockSpec((1, dim), lambda g: (0, 0)),
            ],
            out_specs=pl.BlockSpec((block_b, HW, dim), lambda g: (g, 0, 0)),
            scratch_shapes=[
                pltpu.VMEM((block_b, R, hidden), jnp.float32),
            ],
        ),
        compiler_params=pltpu.CompilerParams(
            dimension_semantics=("parallel",),
            vmem_limit_bytes=vmem_limit),
        cost_estimate=cost,
    )(xp, w1, b1r, dwk, dwbr, w2, b2r)

    return out[:B] if Bp != B else out
